# Initial kernel scaffold; baseline (speedup 1.0000x reference)
import jax
import jax.numpy as jnp
from jax.experimental import pallas as pl


def kernel(x, adj, bi_adj, output, labels_for_lp, W1, b1, W3, b3):
    raise NotImplementedError("write your pallas kernel here")


# speedup vs baseline: 1.0008x; 1.0008x over previous
import jax
import jax.numpy as jnp
from jax.experimental import pallas as pl
from jax.experimental.pallas import tpu as pltpu

N = 4096
NFEAT = 128
NHID = 64
NOUT = 16
BLK = 256
NBLK = N // BLK


def _gcn_kernel(x_ref, adj_ref, bi_ref, lab_ref, w1_ref, b1_ref, w3_ref, b3_ref,
                x3_ref, yhat_ref, masksum_ref,
                adj_c, h_c, s1_c, s3_c):
    p = pl.program_id(0)
    i = pl.program_id(1)

    @pl.when(jnp.logical_and(p == 0, i == 0))
    def _prologue():
        s1 = jnp.dot(x_ref[...].astype(jnp.bfloat16),
                     w1_ref[...].astype(jnp.bfloat16),
                     preferred_element_type=jnp.float32)
        s1_c[...] = s1.astype(jnp.bfloat16)
        rs = jnp.sum(lab_ref[...], axis=1, keepdims=True)
        masksum_ref[...] = (rs > 0.5).astype(jnp.int8)

    @pl.when(p == 0)
    def _pass0():
        ab = adj_ref[...].astype(jnp.bfloat16)
        adj_c[pl.ds(i * BLK, BLK), :] = ab
        hb = jnp.dot(ab, s1_c[...], preferred_element_type=jnp.float32) + b1_ref[...]
        h_c[pl.ds(i * BLK, BLK), :] = jnp.maximum(hb, 0.0).astype(jnp.bfloat16)
        yhat_ref[...] = jnp.dot(bi_ref[...].astype(jnp.bfloat16),
                                lab_ref[...].astype(jnp.bfloat16),
                                preferred_element_type=jnp.float32)

    @pl.when(jnp.logical_and(p == 1, i == 0))
    def _mid():
        s3 = jnp.dot(h_c[...], w3_ref[...].astype(jnp.bfloat16),
                     preferred_element_type=jnp.float32)
        s3_c[...] = s3.astype(jnp.bfloat16)

    @pl.when(p == 1)
    def _pass1():
        x3_ref[...] = jnp.dot(adj_c[pl.ds(i * BLK, BLK), :], s3_c[...],
                              preferred_element_type=jnp.float32) + b3_ref[...]


def kernel(x, adj, bi_adj, output, labels_for_lp, W1, b1, W3, b3):
    del output
    b1r = b1.reshape(1, NHID)
    b3r = b3.reshape(1, NOUT)
    x3, yhat, masksum = pl.pallas_call(
        _gcn_kernel,
        grid=(2, NBLK),
        in_specs=[
            pl.BlockSpec((N, NFEAT), lambda p, i: (0, 0)),
            pl.BlockSpec((BLK, N), lambda p, i: (i + p * (NBLK - 1 - i), 0)),
            pl.BlockSpec((BLK, N), lambda p, i: (i + p * (NBLK - 1 - i), 0)),
            pl.BlockSpec((N, NOUT), lambda p, i: (0, 0)),
            pl.BlockSpec((NFEAT, NHID), lambda p, i: (0, 0)),
            pl.BlockSpec((1, NHID), lambda p, i: (0, 0)),
            pl.BlockSpec((NHID, NOUT), lambda p, i: (0, 0)),
            pl.BlockSpec((1, NOUT), lambda p, i: (0, 0)),
        ],
        out_specs=[
            pl.BlockSpec((BLK, NOUT), lambda p, i: (i * p, 0)),
            pl.BlockSpec((BLK, NOUT), lambda p, i: (i + p * (NBLK - 1 - i), 0)),
            pl.BlockSpec((N, 1), lambda p, i: (0, 0)),
        ],
        out_shape=[
            jax.ShapeDtypeStruct((N, NOUT), jnp.float32),
            jax.ShapeDtypeStruct((N, NOUT), jnp.float32),
            jax.ShapeDtypeStruct((N, 1), jnp.int8),
        ],
        scratch_shapes=[
            pltpu.VMEM((N, N), jnp.bfloat16),
            pltpu.VMEM((N, NHID), jnp.bfloat16),
            pltpu.VMEM((N, NHID), jnp.bfloat16),
            pltpu.VMEM((N, NOUT), jnp.bfloat16),
        ],
        compiler_params=pltpu.CompilerParams(
            dimension_semantics=("arbitrary", "arbitrary"),
        ),
    )(x, adj, bi_adj, labels_for_lp, W1, b1r, W3, b3r)
    mask = masksum[:, 0] > 0
    return (x3, yhat, mask)
